# Initial kernel scaffold; baseline (speedup 1.0000x reference)
import jax
import jax.numpy as jnp
from jax.experimental import pallas as pl


def kernel(x, edge_index, W_embed, W_dec):
    raise NotImplementedError("write your pallas kernel here")


# speedup vs baseline: 5.9724x; 5.9724x over previous
import functools

import jax
import jax.numpy as jnp
from jax import lax
from jax.experimental import pallas as pl
from jax.experimental.pallas import tpu as pltpu
from jax.experimental.pallas import tpu_sc as plsc

_N = 10000
_E = 160000
_IN = 256
_HID = 512
_CLS = 64
_ITERS = 5
_DAMP = 0.5

_NC = 2
_NS = 16
_NW = _NC * _NS
_CH = 256
_NCHUNK = 20
_EPW = _CH * _NCHUNK
_EPAD = _NW * _EPW
_NACC = 10112
_RPT = _NACC // _NS

_mesh = plsc.VectorSubcoreMesh(
    core_axis_name="c", subcore_axis_name="s", num_cores=_NC, num_subcores=_NS
)
_sc_params = pltpu.CompilerParams(use_tc_tiling_on_sc=False)


def _wid():
    return lax.axis_index("c") * _NS + lax.axis_index("s")


@functools.partial(
    pl.kernel,
    out_type=jax.ShapeDtypeStruct((_NC, _NACC), jnp.float32),
    mesh=_mesh,
    compiler_params=_sc_params,
    scratch_types=[
        pltpu.VMEM((_NCHUNK, _CH), jnp.int32),
        pltpu.VMEM((_CH,), jnp.float32),
        pltpu.VMEM_SHARED((_NACC,), jnp.float32),
    ],
)
def _deg(col_hbm, zero_hbm, out_hbm, col_v, ones_v, acc):
    cid = lax.axis_index("c")
    sid = lax.axis_index("s")
    pltpu.sync_copy(col_hbm.at[_wid()], col_v)
    for i in range(_CH // 16):
        ones_v[pl.ds(i * 16, 16)] = jnp.ones((16,), jnp.float32)

    @pl.when(sid == 0)
    def _():
        pltpu.sync_copy(zero_hbm, acc)

    plsc.subcore_barrier()

    @pl.loop(0, _NCHUNK)
    def _(j):
        pltpu.sync_copy(ones_v, acc.at[col_v.at[j]], add=True)

    plsc.subcore_barrier()

    @pl.when(sid == 0)
    def _():
        pltpu.sync_copy(acc, out_hbm.at[cid])


@functools.partial(
    pl.kernel,
    out_type=jax.ShapeDtypeStruct((_NW, _NCHUNK, _CH), jnp.float32),
    mesh=_mesh,
    compiler_params=_sc_params,
    scratch_types=[
        pltpu.VMEM((_NCHUNK, _CH), jnp.int32),
        pltpu.VMEM((_NCHUNK, _CH), jnp.float32),
    ],
)
def _ew(di_hbm, col_hbm, out_hbm, col_v, ew_v):
    w = _wid()
    pltpu.sync_copy(col_hbm.at[w], col_v)

    @pl.loop(0, _NCHUNK)
    def _(j):
        pltpu.sync_copy(di_hbm.at[col_v.at[j]], ew_v.at[j])

    pltpu.sync_copy(ew_v, out_hbm.at[w])


@functools.partial(
    pl.kernel,
    out_type=jax.ShapeDtypeStruct((_NC, _NACC, _CLS), jnp.float32),
    mesh=_mesh,
    compiler_params=_sc_params,
    scratch_types=[
        pltpu.VMEM((_NCHUNK, _CH), jnp.int32),
        pltpu.VMEM((_NCHUNK, _CH), jnp.int32),
        [pltpu.VMEM((_CH, _CLS), jnp.float32)] * 4,
        pltpu.VMEM_SHARED((_NACC, _CLS), jnp.float32),
        [pltpu.SemaphoreType.DMA] * 4,
        [pltpu.SemaphoreType.DMA] * 4,
    ],
)
def _scatter(q_hbm, row_hbm, col_hbm, zero_hbm, out_hbm,
             row_v, col_v, bufs, acc, gsems, ssems):
    cid = lax.axis_index("c")
    sid = lax.axis_index("s")
    w = _wid()
    pltpu.sync_copy(row_hbm.at[w], row_v)
    pltpu.sync_copy(col_hbm.at[w], col_v)
    base = sid * _RPT
    pltpu.sync_copy(zero_hbm.at[pl.ds(base, _RPT)], acc.at[pl.ds(base, _RPT)])
    plsc.subcore_barrier()

    def gather(j, b):
        return pltpu.async_copy(q_hbm.at[row_v.at[j]], bufs[b], gsems[b])

    def scat(j, b):
        return pltpu.async_copy(bufs[b], acc.at[col_v.at[j]], ssems[b],
                                add=True)

    def gather_wait(j, b):
        pltpu.make_async_copy(q_hbm.at[row_v.at[j]], bufs[b], gsems[b]).wait()

    def scat_wait(j, b):
        pltpu.make_async_copy(bufs[b], acc.at[col_v.at[j]], ssems[b]).wait()

    gather(0, 0)
    gather(1, 1)

    @pl.loop(0, _NCHUNK // 4)
    def _(g):
        j0 = g * 4
        for b in range(4):
            j = j0 + b
            b2 = (b + 2) % 4
            gather_wait(j, b)
            scat(j, b)

            @pl.when(j >= 2)
            def _():
                scat_wait(j - 2, b2)

            @pl.when(j + 2 < _NCHUNK)
            def _():
                gather(j + 2, b2)

    scat_wait(_NCHUNK - 2, (_NCHUNK - 2) % 4)
    scat_wait(_NCHUNK - 1, (_NCHUNK - 1) % 4)

    plsc.subcore_barrier()
    pltpu.sync_copy(acc.at[pl.ds(base, _RPT)],
                    out_hbm.at[cid, pl.ds(base, _RPT)])


def _dense_body(x_ref, we_ref, wd_ref, lg_ref, q_ref, xe_ref):
    x1 = jnp.dot(x_ref[...], we_ref[...], preferred_element_type=jnp.float32)
    lg = jnp.dot(x1, wd_ref[...], preferred_element_type=jnp.float32)
    lg_ref[...] = lg
    m = jnp.max(lg, axis=1, keepdims=True)
    e = jnp.exp(lg - m)
    q_ref[...] = e / jnp.sum(e, axis=1, keepdims=True)
    nrm = jnp.sqrt(jnp.sum(x1 * x1, axis=1, keepdims=True))
    xe_ref[...] = x1 / jnp.maximum(nrm, 1e-12)


_BLK = 1000


def _dense(x, we_t, wd_t):
    return pl.pallas_call(
        _dense_body,
        grid=(_N // _BLK,),
        in_specs=[
            pl.BlockSpec((_BLK, _IN), lambda i: (i, 0)),
            pl.BlockSpec((_IN, _HID), lambda i: (0, 0)),
            pl.BlockSpec((_HID, _CLS), lambda i: (0, 0)),
        ],
        out_specs=[
            pl.BlockSpec((_BLK, _CLS), lambda i: (i, 0)),
            pl.BlockSpec((_BLK, _CLS), lambda i: (i, 0)),
            pl.BlockSpec((_BLK, _HID), lambda i: (i, 0)),
        ],
        out_shape=[
            jax.ShapeDtypeStruct((_N, _CLS), jnp.float32),
            jax.ShapeDtypeStruct((_N, _CLS), jnp.float32),
            jax.ShapeDtypeStruct((_N, _HID), jnp.float32),
        ],
    )(x, we_t, wd_t)


def _deginv_body(parts_ref, out_ref):
    d = parts_ref[0, :] + parts_ref[1, :]
    out_ref[...] = jnp.where(d > 0, 1.0 / d, 0.0)


def _deginv(parts):
    return pl.pallas_call(
        _deginv_body,
        out_shape=jax.ShapeDtypeStruct((_NACC,), jnp.float32),
    )(parts)


def _update_body(lg_ref, p0_ref, p1_ref, di_ref, q_ref):
    agg = p0_ref[...] + p1_ref[...]
    s = lg_ref[...] + _DAMP * di_ref[...] * agg
    m = jnp.max(s, axis=1, keepdims=True)
    e = jnp.exp(s - m)
    q_ref[...] = e / jnp.sum(e, axis=1, keepdims=True)


def _update(logits, parts, deginv):
    return pl.pallas_call(
        _update_body,
        grid=(_N // _BLK,),
        in_specs=[
            pl.BlockSpec((_BLK, _CLS), lambda i: (i, 0)),
            pl.BlockSpec((None, _BLK, _CLS), lambda i: (0, i, 0)),
            pl.BlockSpec((None, _BLK, _CLS), lambda i: (1, i, 0)),
            pl.BlockSpec((_BLK, 1), lambda i: (i, 0)),
        ],
        out_specs=pl.BlockSpec((_BLK, _CLS), lambda i: (i, 0)),
        out_shape=jax.ShapeDtypeStruct((_N, _CLS), jnp.float32),
    )(logits, parts, parts, deginv)


def kernel(x, edge_index, W_embed, W_dec):
    row = edge_index[0]
    col = edge_index[1]
    rowp = jnp.concatenate(
        [row, jnp.zeros((_EPAD - _E,), jnp.int32)]).reshape(_NW, _NCHUNK, _CH)
    colp = jnp.concatenate(
        [col, jnp.full((_EPAD - _E,), _N, jnp.int32)]).reshape(_NW, _NCHUNK, _CH)
    zero_n = jnp.zeros((_NACC,), jnp.float32)
    zero_nc = jnp.zeros((_NACC, _CLS), jnp.float32)

    logits, q, x_embed = _dense(x, W_embed.T, W_dec.T)
    degp = _deg(colp, zero_n)
    deginv = _deginv(degp)
    ew_full = _ew(deginv, colp)
    ew = ew_full.reshape(-1)[:_E]
    deginv2 = deginv.reshape(_NACC, 1)

    for _ in range(_ITERS):
        parts = _scatter(q, rowp, colp, zero_nc)
        q = _update(logits, parts, deginv2)

    return (q, x_embed, ew)
